# Initial kernel scaffold; baseline (speedup 1.0000x reference)
import jax
import jax.numpy as jnp
from jax.experimental import pallas as pl


def kernel(x, edge_index, W1, b1, W2, b2):
    raise NotImplementedError("write your pallas kernel here")


# speedup vs baseline: 7.6398x; 7.6398x over previous
import functools

import jax
import jax.numpy as jnp
from jax import lax
from jax.experimental import pallas as pl
from jax.experimental.pallas import tpu as pltpu
from jax.experimental.pallas import tpu_sc as plsc

N = 10000
E = 320000
D = 128

NC, NS, L = 2, 16, 16
NW = NC * NS
CHUNK = 128
N_BLOCKS = 79
N_PAD = N_BLOCKS * 128
CHUNKS_PER_TILE = 79
E_PAD = NW * CHUNKS_PER_TILE * CHUNK
ROWS_PER_TILE = N_PAD // NS

_mesh = plsc.VectorSubcoreMesh(core_axis_name="c", subcore_axis_name="s")


def _tile_ids():
    cid = lax.axis_index("c")
    sid = lax.axis_index("s")
    return cid, sid


@functools.partial(
    pl.kernel,
    out_type=jax.ShapeDtypeStruct((NC * N_PAD, D), jnp.float32),
    mesh=_mesh,
    scratch_types=[
        pltpu.VMEM((1, CHUNK), jnp.int32),
        pltpu.VMEM((1, CHUNK), jnp.int32),
        pltpu.VMEM((CHUNK, D), jnp.float32),
        pltpu.VMEM_SHARED((N_PAD, D), jnp.float32),
        pltpu.SemaphoreType.DMA,
    ],
)
def _sc_aggregate(hp_hbm, src_hbm, dst_hbm, out_hbm,
                  isrc_v, idst_v, rows_v, acc_sh, sem):
    cid, sid = _tile_ids()
    row0 = sid * ROWS_PER_TILE
    rchunks = [(q * CHUNK, min(CHUNK, ROWS_PER_TILE - q * CHUNK))
               for q in range((ROWS_PER_TILE + CHUNK - 1) // CHUNK)]

    def fill_zero(i, _):
        for j in range(D // 16):
            rows_v[i, pl.ds(j * 16, 16)] = jnp.zeros((16,), jnp.float32)
        return _

    lax.fori_loop(0, CHUNK, fill_zero, None)
    for r0, rn in rchunks:
        pltpu.sync_copy(rows_v.at[pl.ds(0, rn)],
                        acc_sh.at[pl.ds(row0 + r0, rn)])
    plsc.subcore_barrier()

    tile_base = (cid * NS + sid) * (CHUNKS_PER_TILE * CHUNK)

    def step(i, _):
        off = tile_base + i * CHUNK
        pltpu.sync_copy(src_hbm.at[pl.ds(off, CHUNK)], isrc_v.at[0])
        pltpu.sync_copy(dst_hbm.at[pl.ds(off, CHUNK)], idst_v.at[0])
        pltpu.async_copy(hp_hbm.at[isrc_v.at[0]], rows_v, sem).wait()
        pltpu.sync_copy(rows_v, acc_sh.at[idst_v.at[0]], add=True)
        return _

    lax.fori_loop(0, CHUNKS_PER_TILE, step, None)
    plsc.subcore_barrier()

    for r0, rn in rchunks:
        pltpu.sync_copy(acc_sh.at[pl.ds(row0 + r0, rn)],
                        rows_v.at[pl.ds(0, rn)])
        pltpu.sync_copy(rows_v.at[pl.ds(0, rn)],
                        out_hbm.at[pl.ds(cid * N_PAD + row0 + r0, rn)])


def _dinv_block(da, db, g):
    deg = da[:, 0:1] + db[:, 0:1] + 1.0
    rows = g * 128 + lax.broadcasted_iota(jnp.int32, (128, 1), 0)
    return jnp.where(rows < N, lax.rsqrt(deg), 0.0)


def _tc_in_body(x_ref, w_ref, da_ref, db_ref, o_ref):
    g = pl.program_id(0)
    dinv = _dinv_block(da_ref[...], db_ref[...], g)
    o_ref[...] = jnp.dot(x_ref[...], w_ref[...],
                         preferred_element_type=jnp.float32) * dinv


def _tc_mid_body(aa_ref, ab_ref, hp_ref, b_ref, w_ref, da_ref, db_ref, o_ref):
    g = pl.program_id(0)
    dinv = _dinv_block(da_ref[...], db_ref[...], g)
    z = jnp.maximum(
        dinv * (aa_ref[...] + ab_ref[...] + hp_ref[...]) + b_ref[...], 0.0)
    o_ref[...] = jnp.dot(z, w_ref[...],
                         preferred_element_type=jnp.float32) * dinv


def _tc_out_body(aa_ref, ab_ref, hp_ref, b_ref, da_ref, db_ref, o_ref):
    g = pl.program_id(0)
    dinv = _dinv_block(da_ref[...], db_ref[...], g)
    o_ref[...] = jnp.maximum(
        dinv * (aa_ref[...] + ab_ref[...] + hp_ref[...]) + b_ref[...], 0.0)


_row_spec = pl.BlockSpec((128, D), lambda g: (g, 0))
_w_spec = pl.BlockSpec((D, D), lambda g: (0, 0))
_b_spec = pl.BlockSpec((1, D), lambda g: (0, 0))
_da_spec = pl.BlockSpec((128, D), lambda g: (g, 0))
_db_spec = pl.BlockSpec((128, D), lambda g: (N_BLOCKS + g, 0))
_out_struct = jax.ShapeDtypeStruct((N_PAD, D), jnp.float32)


def _tc_in(x_p, w, deg2):
    return pl.pallas_call(
        _tc_in_body, grid=(N_BLOCKS,),
        in_specs=[_row_spec, _w_spec, _da_spec, _db_spec],
        out_specs=_row_spec, out_shape=_out_struct,
    )(x_p, w, deg2, deg2)


def _tc_mid(acc, hp, b, w, deg2):
    return pl.pallas_call(
        _tc_mid_body, grid=(N_BLOCKS,),
        in_specs=[_row_spec,
                  pl.BlockSpec((128, D), lambda g: (N_BLOCKS + g, 0)),
                  _row_spec, _b_spec, _w_spec, _da_spec, _db_spec],
        out_specs=_row_spec, out_shape=_out_struct,
    )(acc, acc, hp, b, w, deg2, deg2)


def _tc_out(acc, hp, b, deg2):
    return pl.pallas_call(
        _tc_out_body, grid=(N_BLOCKS,),
        in_specs=[_row_spec,
                  pl.BlockSpec((128, D), lambda g: (N_BLOCKS + g, 0)),
                  _row_spec, _b_spec, _da_spec, _db_spec],
        out_specs=_row_spec, out_shape=_out_struct,
    )(acc, acc, hp, b, deg2, deg2)


def kernel(x, edge_index, W1, b1, W2, b2):
    pad_e = E_PAD - E
    src_p = jnp.concatenate(
        [edge_index[0], jnp.full((pad_e,), N, jnp.int32)])
    dst_p = jnp.concatenate(
        [edge_index[1], jnp.full((pad_e,), N, jnp.int32)])
    x_p = jnp.concatenate([x, jnp.zeros((N_PAD - N, D), jnp.float32)])
    b1r = b1.reshape(1, D)
    b2r = b2.reshape(1, D)

    deg2 = _sc_aggregate(jnp.ones((N_PAD, D), jnp.float32), dst_p, dst_p)
    h1p = _tc_in(x_p, W1, deg2)
    acc1 = _sc_aggregate(h1p, src_p, dst_p)
    h2p = _tc_mid(acc1, h1p, b1r, W2, deg2)
    acc2 = _sc_aggregate(h2p, src_p, dst_p)
    out = _tc_out(acc2, h2p, b2r, deg2)
    return out[:N]
